# baseline (speedup 1.0000x reference)
import functools

import jax
import jax.numpy as jnp
from jax import lax
from jax.experimental import pallas as pl
from jax.experimental.pallas import tpu as pltpu
from jax.experimental.pallas import tpu_sc as plsc

N = 10000
N_PAD = 10240
D = 128
Q = 64
NNZ = 320000
BATCH = 1024
TEMP = 0.2
LAMBDA_1 = 0.2
LAMBDA_2 = 1e-07

NUM_CORES = 2
NUM_SUBCORES = 16
EDGES_PER_TILE = NNZ // NUM_SUBCORES
EDGE_BLK = 80
NBLK = EDGES_PER_TILE // EDGE_BLK
ROWS_PER_TILE = N_PAD // NUM_SUBCORES

_VMESH = plsc.VectorSubcoreMesh(core_axis_name="c", subcore_axis_name="s")


def _spmm_stage(table, init, gidx_all, sidx_all, val):
    @functools.partial(
        pl.kernel,
        out_type=jax.ShapeDtypeStruct((NUM_CORES * N_PAD, D), jnp.float32),
        mesh=_VMESH,
        scratch_types=[
            pltpu.VMEM_SHARED((N_PAD, D), jnp.float32),
            pltpu.VMEM((2, EDGE_BLK), jnp.int32),
            pltpu.VMEM((2, EDGE_BLK), jnp.int32),
            pltpu.VMEM((2, EDGE_BLK), jnp.float32),
            pltpu.VMEM((2, EDGE_BLK, D), jnp.float32),
            pltpu.SemaphoreType.DMA((2,)),
            pltpu.SemaphoreType.DMA((2,)),
            pltpu.SemaphoreType.DMA((2,)),
            pltpu.SemaphoreType.DMA((2,)),
            pltpu.SemaphoreType.DMA((2,)),
        ],
    )
    def kern(table_hbm, init_hbm, gidx_hbm, sidx_hbm, val_hbm, out_hbm,
             acc, gidx_v, sidx_v, val_v, rows_v,
             g_sem, w_sem, i_sem, s_sem, v_sem):
        c = lax.axis_index("c")
        s = lax.axis_index("s")
        r0 = s * ROWS_PER_TILE
        pltpu.sync_copy(init_hbm.at[pl.ds(c * N_PAD + r0, ROWS_PER_TILE)],
                        acc.at[pl.ds(r0, ROWS_PER_TILE)])
        plsc.subcore_barrier()

        e0 = s * EDGES_PER_TILE
        ec0 = c * NNZ + e0

        def issue_gidxval(b, k):
            base = e0 + b * EDGE_BLK
            cbase = ec0 + b * EDGE_BLK
            pltpu.async_copy(gidx_hbm.at[pl.ds(cbase, EDGE_BLK)],
                             gidx_v.at[k], i_sem.at[k])
            pltpu.async_copy(val_hbm.at[pl.ds(base, EDGE_BLK)],
                             val_v.at[k], v_sem.at[k])

        def issue_gather(k):
            pltpu.async_copy(table_hbm.at[gidx_v.at[k]], rows_v.at[k],
                             g_sem.at[k])

        def wait_gidx(k):
            pltpu.make_async_copy(gidx_hbm.at[pl.ds(0, EDGE_BLK)],
                                  gidx_v.at[k], i_sem.at[k]).wait()

        def wait_val(k):
            pltpu.make_async_copy(val_hbm.at[pl.ds(0, EDGE_BLK)],
                                  val_v.at[k], v_sem.at[k]).wait()

        def wait_gather(k):
            pltpu.make_async_copy(table_hbm.at[gidx_v.at[k]], rows_v.at[k],
                                  g_sem.at[k]).wait()

        def wait_scatter(k):
            pltpu.make_async_copy(rows_v.at[k], acc.at[sidx_v.at[k]],
                                  w_sem.at[k]).wait()

        def body(b, k, first=False, next_idx=True, next_gather=True):
            cbase = ec0 + b * EDGE_BLK
            pltpu.async_copy(sidx_hbm.at[pl.ds(cbase, EDGE_BLK)],
                             sidx_v.at[k], s_sem.at[k])
            wait_gather(k)
            wait_val(k)
            @plsc.parallel_loop(0, EDGE_BLK // 16, unroll=5)
            def _grp(g):
                vv = val_v[k, pl.ds(g * 16, 16)]
                e0g = g * 16
                for j in range(16):
                    vb = lax.broadcast(vv[j], (16,))
                    for kk in range(D // 16):
                        sl = pl.ds(kk * 16, 16)
                        rows_v[k, e0g + j, sl] = rows_v[k, e0g + j, sl] * vb

            if next_idx:
                issue_gidxval(b + 2, k)
            pltpu.make_async_copy(sidx_hbm.at[pl.ds(0, EDGE_BLK)],
                                  sidx_v.at[k], s_sem.at[k]).wait()
            pltpu.async_copy(rows_v.at[k], acc.at[sidx_v.at[k]],
                             w_sem.at[k], add=True)
            if next_gather:
                if not first:
                    wait_scatter(1 - k)
                wait_gidx(1 - k)
                issue_gather(1 - k)

        issue_gidxval(0, 0)
        issue_gidxval(1, 1)
        wait_gidx(0)
        issue_gather(0)

        body(0, 0, first=True)
        body(1, 1)

        @pl.loop(1, NBLK // 2 - 1)
        def _pair(t):
            body(2 * t, 0)
            body(2 * t + 1, 1)

        body(NBLK - 2, 0, next_idx=False)
        body(NBLK - 1, 1, next_idx=False, next_gather=False)
        wait_scatter(0)
        wait_scatter(1)

        plsc.subcore_barrier()
        pltpu.sync_copy(acc.at[pl.ds(r0, ROWS_PER_TILE)],
                        out_hbm.at[pl.ds(c * N_PAD + r0, ROWS_PER_TILE)])

    return kern(table, init, gidx_all, sidx_all, val)


def _batch_gather(g_table, e_table, idx_g, idx_e):
    ng = idx_g.shape[0] // (NUM_CORES * NUM_SUBCORES)
    ne = idx_e.shape[0] // (NUM_CORES * NUM_SUBCORES)

    @functools.partial(
        pl.kernel,
        out_type=(
            jax.ShapeDtypeStruct((idx_g.shape[0], D), jnp.float32),
            jax.ShapeDtypeStruct((idx_e.shape[0], D), jnp.float32),
        ),
        mesh=_VMESH,
        scratch_types=[
            pltpu.VMEM((ng,), jnp.int32),
            pltpu.VMEM((ne,), jnp.int32),
            pltpu.VMEM((ng, D), jnp.float32),
            pltpu.VMEM((ne, D), jnp.float32),
            pltpu.SemaphoreType.DMA,
        ],
    )
    def kern(gt_hbm, et_hbm, ig_hbm, ie_hbm, og_hbm, oe_hbm,
             ig_v, ie_v, rg_v, re_v, sem):
        w = lax.axis_index("c") * NUM_SUBCORES + lax.axis_index("s")
        pltpu.sync_copy(ig_hbm.at[pl.ds(w * ng, ng)], ig_v)
        pltpu.async_copy(gt_hbm.at[ig_v], rg_v, sem).wait()
        pltpu.sync_copy(rg_v, og_hbm.at[pl.ds(w * ng, ng)])
        pltpu.sync_copy(ie_hbm.at[pl.ds(w * ne, ne)], ie_v)
        pltpu.async_copy(et_hbm.at[ie_v], re_v, sem).wait()
        pltpu.sync_copy(re_v, oe_hbm.at[pl.ds(w * ne, ne)])

    return kern(g_table, e_table, idx_g, idx_e)


_PCH = 2048


def _p_kernel(w_stack, s_out):
    nch = N_PAD // _PCH

    def body(w_ref, s_ref, p_ref):
        i = pl.program_id(1)

        @pl.when(i == 0)
        def _():
            p_ref[...] = jnp.zeros_like(p_ref)

        p_ref[0] += lax.dot_general(
            w_ref[0], s_ref[...], (((1,), (0,)), ((), ())),
            preferred_element_type=jnp.float32)

    return pl.pallas_call(
        body,
        grid=(2, nch),
        in_specs=[
            pl.BlockSpec((1, Q, _PCH), lambda g, i: (g, 0, i)),
            pl.BlockSpec((_PCH, D), lambda g, i: ((1 - g) * nch + i, 0)),
        ],
        out_specs=pl.BlockSpec((1, Q, D), lambda g, i: (g, 0, 0)),
        out_shape=jax.ShapeDtypeStruct((2, Q, D), jnp.float32),
    )(w_stack, s_out)


_GCH = 2048


def _g_kernel(e0_stack, ms_stack, p_stack):
    nch = N_PAD // _GCH

    def body(e0_ref, ms_ref, p_ref, g_ref, reg_ref):
        g = pl.program_id(0)
        i = pl.program_id(1)
        e0 = e0_ref[0]
        g_ref[...] = e0 + lax.dot_general(
            ms_ref[0], p_ref[0], (((1,), (0,)), ((), ())),
            preferred_element_type=jnp.float32)
        part = jnp.sum(e0 * e0)

        @pl.when((g == 0) & (i == 0))
        def _():
            reg_ref[0, 0] = part * LAMBDA_2

        @pl.when((g > 0) | (i > 0))
        def _():
            reg_ref[0, 0] += part * LAMBDA_2

    return pl.pallas_call(
        body,
        grid=(2, nch),
        in_specs=[
            pl.BlockSpec((1, _GCH, D), lambda g, i: (g, i, 0)),
            pl.BlockSpec((1, _GCH, Q), lambda g, i: (g, i, 0)),
            pl.BlockSpec((1, Q, D), lambda g, i: (g, 0, 0)),
        ],
        out_specs=[
            pl.BlockSpec((_GCH, D), lambda g, i: (g * nch + i, 0)),
            pl.BlockSpec((1, 1), lambda g, i: (0, 0),
                         memory_space=pltpu.SMEM),
        ],
        out_shape=[
            jax.ShapeDtypeStruct((2 * N_PAD, D), jnp.float32),
            jax.ShapeDtypeStruct((1, 1), jnp.float32),
        ],
    )(e0_stack, ms_stack, p_stack)


_LCH = 1024
_NPADROWS = float(N_PAD - N)


def _loss_kernel(e_tables, bg, be, reg):
    nsteps = N_PAD // _LCH

    def body(eu_ref, ei_ref, bg_ref, be_ref, reg_ref,
             loss_ref, lr_ref, ls_ref, accu, acci):
        j = pl.program_id(0)
        inv_t = 1.0 / TEMP
        gub = bg_ref[pl.ds(0, BATCH), :] * inv_t
        gib = bg_ref[pl.ds(BATCH, BATCH), :] * inv_t
        lu = lax.dot_general(gub, eu_ref[...], (((1,), (1,)), ((), ())),
                             preferred_element_type=jnp.float32)
        li = lax.dot_general(gib, ei_ref[...], (((1,), (1,)), ((), ())),
                             preferred_element_type=jnp.float32)
        su = jnp.sum(jnp.exp(lu), axis=1)
        si = jnp.sum(jnp.exp(li), axis=1)

        @pl.when(j == 0)
        def _():
            accu[...] = su
            acci[...] = si

        @pl.when(j > 0)
        def _():
            accu[...] += su
            acci[...] += si

        @pl.when(j == nsteps - 1)
        def _():
            eub = be_ref[pl.ds(0, BATCH), :]
            eip = be_ref[pl.ds(BATCH, BATCH), :]
            ein = be_ref[pl.ds(2 * BATCH, BATCH), :]
            eib = be_ref[pl.ds(3 * BATCH, BATCH), :]
            neg_score = (jnp.mean(jnp.log(accu[...] - _NPADROWS + 1e-8))
                         + jnp.mean(jnp.log(acci[...] - _NPADROWS + 1e-8)))
            pos_score = (jnp.mean(jnp.clip(jnp.sum(gub * eub, axis=1), -5.0, 5.0))
                         + jnp.mean(jnp.clip(jnp.sum(gib * eib, axis=1), -5.0, 5.0)))
            loss_s = neg_score - pos_score
            d = jnp.sum(eub * (eip - ein), axis=1)
            loss_r = jnp.mean(jnp.log(1.0 + jnp.exp(-d)))
            lr_ref[0, 0] = loss_r
            ls_ref[0, 0] = LAMBDA_1 * loss_s
            loss_ref[0, 0] = loss_r + LAMBDA_1 * loss_s + reg_ref[0, 0]

    return pl.pallas_call(
        body,
        grid=(nsteps,),
        in_specs=[
            pl.BlockSpec((_LCH, D), lambda j: (j, 0)),
            pl.BlockSpec((_LCH, D), lambda j: (N_PAD // _LCH + j, 0)),
            pl.BlockSpec((2 * BATCH, D), lambda j: (0, 0)),
            pl.BlockSpec((4 * BATCH, D), lambda j: (0, 0)),
            pl.BlockSpec((1, 1), lambda j: (0, 0), memory_space=pltpu.SMEM),
        ],
        out_specs=[
            pl.BlockSpec((1, 1), lambda j: (0, 0), memory_space=pltpu.SMEM),
            pl.BlockSpec((1, 1), lambda j: (0, 0), memory_space=pltpu.SMEM),
            pl.BlockSpec((1, 1), lambda j: (0, 0), memory_space=pltpu.SMEM),
        ],
        out_shape=[
            jax.ShapeDtypeStruct((1, 1), jnp.float32),
            jax.ShapeDtypeStruct((1, 1), jnp.float32),
            jax.ShapeDtypeStruct((1, 1), jnp.float32),
        ],
        scratch_shapes=[
            pltpu.VMEM((BATCH,), jnp.float32),
            pltpu.VMEM((BATCH,), jnp.float32),
        ],
    )(e_tables, e_tables, bg, be, reg)


def kernel(uids, iids, pos, neg, E_u_0, E_i_0, u_mul_s, v_mul_s, ut, vt,
           adj_val, adj_row, adj_col):
    pad = ((0, N_PAD - N), (0, 0))
    eu0p = jnp.pad(E_u_0, pad)
    ei0p = jnp.pad(E_i_0, pad)
    e0_stack = jnp.stack([eu0p, ei0p])
    t0 = jnp.concatenate([eu0p, ei0p], axis=0)
    w_stack = jnp.stack([jnp.pad(vt, ((0, 0), (0, N_PAD - N))),
                         jnp.pad(ut, ((0, 0), (0, N_PAD - N)))])
    ms_stack = jnp.stack([jnp.pad(u_mul_s, pad), jnp.pad(v_mul_s, pad)])

    row = adj_row.astype(jnp.int32)
    col = adj_col.astype(jnp.int32)
    gidx_all = jnp.concatenate([col + N_PAD, row])
    sidx_all = jnp.concatenate([row, col])

    s_out = _spmm_stage(t0, t0, gidx_all, sidx_all, adj_val)
    e_out = _spmm_stage(s_out, t0, gidx_all, sidx_all, adj_val)

    p_stack = _p_kernel(w_stack, s_out)
    g_stack, reg = _g_kernel(e0_stack, ms_stack, p_stack)

    uidsi = uids.astype(jnp.int32)
    iidsi = iids.astype(jnp.int32)
    idx_g = jnp.concatenate([uidsi, iidsi + N_PAD])
    idx_e = jnp.concatenate([uidsi, pos.astype(jnp.int32) + N_PAD,
                             neg.astype(jnp.int32) + N_PAD, iidsi + N_PAD])
    bg, be = _batch_gather(g_stack, e_out, idx_g, idx_e)

    loss, lr, ls = _loss_kernel(e_out, bg, be, reg)
    return (loss[0, 0].reshape(()), lr[0, 0].reshape(()), ls[0, 0].reshape(()))


# speedup vs baseline: 1.1468x; 1.1468x over previous
import functools

import jax
import jax.numpy as jnp
from jax import lax
from jax.experimental import pallas as pl
from jax.experimental.pallas import tpu as pltpu
from jax.experimental.pallas import tpu_sc as plsc

N = 10000
N_PAD = 10240
D = 128
Q = 64
NNZ = 320000
BATCH = 1024
TEMP = 0.2
LAMBDA_1 = 0.2
LAMBDA_2 = 1e-07

NUM_CORES = 2
NUM_SUBCORES = 16
EDGES_PER_TILE = NNZ // NUM_SUBCORES
EDGE_BLK = 128
NBLK = EDGES_PER_TILE // EDGE_BLK
TAIL = EDGES_PER_TILE - NBLK * EDGE_BLK
ROWS_PER_TILE = N_PAD // NUM_SUBCORES

_VMESH = plsc.VectorSubcoreMesh(core_axis_name="c", subcore_axis_name="s")


def _spmm_stage(table, init, gidx_all, sidx_all, val):
    @functools.partial(
        pl.kernel,
        out_type=jax.ShapeDtypeStruct((NUM_CORES * N_PAD, D), jnp.float32),
        mesh=_VMESH,
        scratch_types=[
            pltpu.VMEM_SHARED((N_PAD, D), jnp.float32),
            pltpu.VMEM((2, EDGE_BLK), jnp.int32),
            pltpu.VMEM((2, EDGE_BLK), jnp.int32),
            pltpu.VMEM((2, EDGE_BLK), jnp.float32),
            pltpu.VMEM((2, EDGE_BLK, D), jnp.float32),
            pltpu.VMEM((TAIL,), jnp.int32),
            pltpu.VMEM((TAIL,), jnp.int32),
            pltpu.VMEM((TAIL,), jnp.float32),
            pltpu.VMEM((TAIL, D), jnp.float32),
            pltpu.SemaphoreType.DMA((2,)),
            pltpu.SemaphoreType.DMA((2,)),
            pltpu.SemaphoreType.DMA((2,)),
            pltpu.SemaphoreType.DMA((2,)),
            pltpu.SemaphoreType.DMA((2,)),
        ],
    )
    def kern(table_hbm, init_hbm, gidx_hbm, sidx_hbm, val_hbm, out_hbm,
             acc, gidx_v, sidx_v, val_v, rows_v,
             t_gidx, t_sidx, t_val, t_rows,
             g_sem, w_sem, i_sem, s_sem, v_sem):
        c = lax.axis_index("c")
        s = lax.axis_index("s")
        r0 = s * ROWS_PER_TILE
        pltpu.sync_copy(init_hbm.at[pl.ds(c * N_PAD + r0, ROWS_PER_TILE)],
                        acc.at[pl.ds(r0, ROWS_PER_TILE)])
        plsc.subcore_barrier()

        e0 = s * EDGES_PER_TILE
        ec0 = c * NNZ + e0

        def issue_gidxval(b, k):
            base = e0 + b * EDGE_BLK
            cbase = ec0 + b * EDGE_BLK
            pltpu.async_copy(gidx_hbm.at[pl.ds(cbase, EDGE_BLK)],
                             gidx_v.at[k], i_sem.at[k])
            pltpu.async_copy(val_hbm.at[pl.ds(base, EDGE_BLK)],
                             val_v.at[k], v_sem.at[k])

        def issue_gather(k):
            pltpu.async_copy(table_hbm.at[gidx_v.at[k]], rows_v.at[k],
                             g_sem.at[k])

        def wait_gidx(k):
            pltpu.make_async_copy(gidx_hbm.at[pl.ds(0, EDGE_BLK)],
                                  gidx_v.at[k], i_sem.at[k]).wait()

        def wait_val(k):
            pltpu.make_async_copy(val_hbm.at[pl.ds(0, EDGE_BLK)],
                                  val_v.at[k], v_sem.at[k]).wait()

        def wait_gather(k):
            pltpu.make_async_copy(table_hbm.at[gidx_v.at[k]], rows_v.at[k],
                                  g_sem.at[k]).wait()

        def wait_scatter(k):
            pltpu.make_async_copy(rows_v.at[k], acc.at[sidx_v.at[k]],
                                  w_sem.at[k]).wait()

        def body(b, k, first=False, next_idx=True, next_gather=True):
            cbase = ec0 + b * EDGE_BLK
            pltpu.async_copy(sidx_hbm.at[pl.ds(cbase, EDGE_BLK)],
                             sidx_v.at[k], s_sem.at[k])
            wait_gather(k)
            wait_val(k)
            @pl.loop(0, EDGE_BLK // 16)
            def _grp(g):
                vv = val_v[k, pl.ds(g * 16, 16)]
                e0g = g * 16
                for j in range(16):
                    v = vv[j]
                    for kk in range(D // 16):
                        sl = pl.ds(kk * 16, 16)
                        rows_v[k, e0g + j, sl] = rows_v[k, e0g + j, sl] * v

            if next_idx:
                issue_gidxval(b + 2, k)
            pltpu.make_async_copy(sidx_hbm.at[pl.ds(0, EDGE_BLK)],
                                  sidx_v.at[k], s_sem.at[k]).wait()
            pltpu.async_copy(rows_v.at[k], acc.at[sidx_v.at[k]],
                             w_sem.at[k], add=True)
            if next_gather:
                if not first:
                    wait_scatter(1 - k)
                wait_gidx(1 - k)
                issue_gather(1 - k)

        issue_gidxval(0, 0)
        issue_gidxval(1, 1)
        wait_gidx(0)
        issue_gather(0)

        body(0, 0, first=True)
        body(1, 1)

        @pl.loop(1, NBLK // 2 - 1)
        def _pair(t):
            body(2 * t, 0)
            body(2 * t + 1, 1)

        body(NBLK - 2, 0, next_idx=False)
        body(NBLK - 1, 1, next_idx=False, next_gather=False)
        wait_scatter(0)
        wait_scatter(1)

        tbase = e0 + NBLK * EDGE_BLK
        ctbase = ec0 + NBLK * EDGE_BLK
        pltpu.sync_copy(gidx_hbm.at[pl.ds(ctbase, TAIL)], t_gidx)
        pltpu.sync_copy(sidx_hbm.at[pl.ds(ctbase, TAIL)], t_sidx)
        pltpu.sync_copy(val_hbm.at[pl.ds(tbase, TAIL)], t_val)
        pltpu.async_copy(table_hbm.at[t_gidx], t_rows, g_sem.at[0]).wait()

        @pl.loop(0, TAIL // 16)
        def _tgrp(g):
            vv = t_val[pl.ds(g * 16, 16)]
            e0g = g * 16
            for j in range(16):
                v = vv[j]
                for kk in range(D // 16):
                    sl = pl.ds(kk * 16, 16)
                    t_rows[e0g + j, sl] = t_rows[e0g + j, sl] * v

        pltpu.sync_copy(t_rows, acc.at[t_sidx], add=True)

        plsc.subcore_barrier()
        pltpu.sync_copy(acc.at[pl.ds(r0, ROWS_PER_TILE)],
                        out_hbm.at[pl.ds(c * N_PAD + r0, ROWS_PER_TILE)])

    return kern(table, init, gidx_all, sidx_all, val)


def _batch_gather(g_table, e_table, idx_g, idx_e):
    ng = idx_g.shape[0] // (NUM_CORES * NUM_SUBCORES)
    ne = idx_e.shape[0] // (NUM_CORES * NUM_SUBCORES)

    @functools.partial(
        pl.kernel,
        out_type=(
            jax.ShapeDtypeStruct((idx_g.shape[0], D), jnp.float32),
            jax.ShapeDtypeStruct((idx_e.shape[0], D), jnp.float32),
        ),
        mesh=_VMESH,
        scratch_types=[
            pltpu.VMEM((ng,), jnp.int32),
            pltpu.VMEM((ne,), jnp.int32),
            pltpu.VMEM((ng, D), jnp.float32),
            pltpu.VMEM((ne, D), jnp.float32),
            pltpu.SemaphoreType.DMA,
        ],
    )
    def kern(gt_hbm, et_hbm, ig_hbm, ie_hbm, og_hbm, oe_hbm,
             ig_v, ie_v, rg_v, re_v, sem):
        w = lax.axis_index("c") * NUM_SUBCORES + lax.axis_index("s")
        pltpu.sync_copy(ig_hbm.at[pl.ds(w * ng, ng)], ig_v)
        pltpu.async_copy(gt_hbm.at[ig_v], rg_v, sem).wait()
        pltpu.sync_copy(rg_v, og_hbm.at[pl.ds(w * ng, ng)])
        pltpu.sync_copy(ie_hbm.at[pl.ds(w * ne, ne)], ie_v)
        pltpu.async_copy(et_hbm.at[ie_v], re_v, sem).wait()
        pltpu.sync_copy(re_v, oe_hbm.at[pl.ds(w * ne, ne)])

    return kern(g_table, e_table, idx_g, idx_e)


_PCH = 2048


def _p_kernel(w_stack, s_out):
    nch = N_PAD // _PCH

    def body(w_ref, s_ref, p_ref):
        i = pl.program_id(1)

        @pl.when(i == 0)
        def _():
            p_ref[...] = jnp.zeros_like(p_ref)

        p_ref[0] += lax.dot_general(
            w_ref[0], s_ref[...], (((1,), (0,)), ((), ())),
            preferred_element_type=jnp.float32)

    return pl.pallas_call(
        body,
        grid=(2, nch),
        in_specs=[
            pl.BlockSpec((1, Q, _PCH), lambda g, i: (g, 0, i)),
            pl.BlockSpec((_PCH, D), lambda g, i: ((1 - g) * nch + i, 0)),
        ],
        out_specs=pl.BlockSpec((1, Q, D), lambda g, i: (g, 0, 0)),
        out_shape=jax.ShapeDtypeStruct((2, Q, D), jnp.float32),
    )(w_stack, s_out)


_GCH = 2048


def _g_kernel(e0_stack, ms_stack, p_stack):
    nch = N_PAD // _GCH

    def body(e0_ref, ms_ref, p_ref, g_ref, reg_ref):
        g = pl.program_id(0)
        i = pl.program_id(1)
        e0 = e0_ref[0]
        g_ref[...] = e0 + lax.dot_general(
            ms_ref[0], p_ref[0], (((1,), (0,)), ((), ())),
            preferred_element_type=jnp.float32)
        part = jnp.sum(e0 * e0)

        @pl.when((g == 0) & (i == 0))
        def _():
            reg_ref[0, 0] = part * LAMBDA_2

        @pl.when((g > 0) | (i > 0))
        def _():
            reg_ref[0, 0] += part * LAMBDA_2

    return pl.pallas_call(
        body,
        grid=(2, nch),
        in_specs=[
            pl.BlockSpec((1, _GCH, D), lambda g, i: (g, i, 0)),
            pl.BlockSpec((1, _GCH, Q), lambda g, i: (g, i, 0)),
            pl.BlockSpec((1, Q, D), lambda g, i: (g, 0, 0)),
        ],
        out_specs=[
            pl.BlockSpec((_GCH, D), lambda g, i: (g * nch + i, 0)),
            pl.BlockSpec((1, 1), lambda g, i: (0, 0),
                         memory_space=pltpu.SMEM),
        ],
        out_shape=[
            jax.ShapeDtypeStruct((2 * N_PAD, D), jnp.float32),
            jax.ShapeDtypeStruct((1, 1), jnp.float32),
        ],
    )(e0_stack, ms_stack, p_stack)


_LCH = 1024
_NPADROWS = float(N_PAD - N)


def _loss_kernel(e_tables, bg, be, reg):
    nsteps = N_PAD // _LCH

    def body(eu_ref, ei_ref, bg_ref, be_ref, reg_ref,
             loss_ref, lr_ref, ls_ref, accu, acci):
        j = pl.program_id(0)
        inv_t = 1.0 / TEMP
        gub = bg_ref[pl.ds(0, BATCH), :] * inv_t
        gib = bg_ref[pl.ds(BATCH, BATCH), :] * inv_t
        lu = lax.dot_general(gub, eu_ref[...], (((1,), (1,)), ((), ())),
                             preferred_element_type=jnp.float32)
        li = lax.dot_general(gib, ei_ref[...], (((1,), (1,)), ((), ())),
                             preferred_element_type=jnp.float32)
        su = jnp.sum(jnp.exp(lu), axis=1)
        si = jnp.sum(jnp.exp(li), axis=1)

        @pl.when(j == 0)
        def _():
            accu[...] = su
            acci[...] = si

        @pl.when(j > 0)
        def _():
            accu[...] += su
            acci[...] += si

        @pl.when(j == nsteps - 1)
        def _():
            eub = be_ref[pl.ds(0, BATCH), :]
            eip = be_ref[pl.ds(BATCH, BATCH), :]
            ein = be_ref[pl.ds(2 * BATCH, BATCH), :]
            eib = be_ref[pl.ds(3 * BATCH, BATCH), :]
            neg_score = (jnp.mean(jnp.log(accu[...] - _NPADROWS + 1e-8))
                         + jnp.mean(jnp.log(acci[...] - _NPADROWS + 1e-8)))
            pos_score = (jnp.mean(jnp.clip(jnp.sum(gub * eub, axis=1), -5.0, 5.0))
                         + jnp.mean(jnp.clip(jnp.sum(gib * eib, axis=1), -5.0, 5.0)))
            loss_s = neg_score - pos_score
            d = jnp.sum(eub * (eip - ein), axis=1)
            loss_r = jnp.mean(jnp.log(1.0 + jnp.exp(-d)))
            lr_ref[0, 0] = loss_r
            ls_ref[0, 0] = LAMBDA_1 * loss_s
            loss_ref[0, 0] = loss_r + LAMBDA_1 * loss_s + reg_ref[0, 0]

    return pl.pallas_call(
        body,
        grid=(nsteps,),
        in_specs=[
            pl.BlockSpec((_LCH, D), lambda j: (j, 0)),
            pl.BlockSpec((_LCH, D), lambda j: (N_PAD // _LCH + j, 0)),
            pl.BlockSpec((2 * BATCH, D), lambda j: (0, 0)),
            pl.BlockSpec((4 * BATCH, D), lambda j: (0, 0)),
            pl.BlockSpec((1, 1), lambda j: (0, 0), memory_space=pltpu.SMEM),
        ],
        out_specs=[
            pl.BlockSpec((1, 1), lambda j: (0, 0), memory_space=pltpu.SMEM),
            pl.BlockSpec((1, 1), lambda j: (0, 0), memory_space=pltpu.SMEM),
            pl.BlockSpec((1, 1), lambda j: (0, 0), memory_space=pltpu.SMEM),
        ],
        out_shape=[
            jax.ShapeDtypeStruct((1, 1), jnp.float32),
            jax.ShapeDtypeStruct((1, 1), jnp.float32),
            jax.ShapeDtypeStruct((1, 1), jnp.float32),
        ],
        scratch_shapes=[
            pltpu.VMEM((BATCH,), jnp.float32),
            pltpu.VMEM((BATCH,), jnp.float32),
        ],
    )(e_tables, e_tables, bg, be, reg)


def kernel(uids, iids, pos, neg, E_u_0, E_i_0, u_mul_s, v_mul_s, ut, vt,
           adj_val, adj_row, adj_col):
    pad = ((0, N_PAD - N), (0, 0))
    eu0p = jnp.pad(E_u_0, pad)
    ei0p = jnp.pad(E_i_0, pad)
    e0_stack = jnp.stack([eu0p, ei0p])
    t0 = jnp.concatenate([eu0p, ei0p], axis=0)
    w_stack = jnp.stack([jnp.pad(vt, ((0, 0), (0, N_PAD - N))),
                         jnp.pad(ut, ((0, 0), (0, N_PAD - N)))])
    ms_stack = jnp.stack([jnp.pad(u_mul_s, pad), jnp.pad(v_mul_s, pad)])

    row = adj_row.astype(jnp.int32)
    col = adj_col.astype(jnp.int32)
    gidx_all = jnp.concatenate([col + N_PAD, row])
    sidx_all = jnp.concatenate([row, col])

    s_out = _spmm_stage(t0, t0, gidx_all, sidx_all, adj_val)
    e_out = _spmm_stage(s_out, t0, gidx_all, sidx_all, adj_val)

    p_stack = _p_kernel(w_stack, s_out)
    g_stack, reg = _g_kernel(e0_stack, ms_stack, p_stack)

    uidsi = uids.astype(jnp.int32)
    iidsi = iids.astype(jnp.int32)
    idx_g = jnp.concatenate([uidsi, iidsi + N_PAD])
    idx_e = jnp.concatenate([uidsi, pos.astype(jnp.int32) + N_PAD,
                             neg.astype(jnp.int32) + N_PAD, iidsi + N_PAD])
    bg, be = _batch_gather(g_stack, e_out, idx_g, idx_e)

    loss, lr, ls = _loss_kernel(e_out, bg, be, reg)
    return (loss[0, 0].reshape(()), lr[0, 0].reshape(()), ls[0, 0].reshape(()))
